# baseline (speedup 1.0000x reference)
import functools

import jax
import jax.numpy as jnp
from jax import lax
from jax.experimental import pallas as pl
from jax.experimental.pallas import tpu as pltpu
from jax.experimental.pallas import tpu_sc as plsc

N_VOCAB = 1000000
N_EMBED = 32
BATCH = 16384

_info = plsc.get_sparse_core_info()
_NC = _info.num_cores
_NS = _info.num_subcores
_NW = _NC * _NS
_B_PER_W = BATCH // _NW
_CHUNK = 128
_N_CHUNK = _B_PER_W // _CHUNK


@functools.partial(
    pl.kernel,
    mesh=plsc.VectorSubcoreMesh(core_axis_name="c", subcore_axis_name="s"),
    out_type=[
        jax.ShapeDtypeStruct((BATCH, N_EMBED), jnp.float32),
        jax.ShapeDtypeStruct((BATCH, N_EMBED), jnp.float32),
        jax.ShapeDtypeStruct((BATCH, N_EMBED), jnp.float32),
    ],
    scratch_types=[
        pltpu.VMEM((_B_PER_W,), jnp.int32),
        pltpu.VMEM((_B_PER_W,), jnp.int32),
        pltpu.VMEM((_B_PER_W,), jnp.int32),
        pltpu.VMEM((_B_PER_W, N_EMBED), jnp.float32),
        pltpu.VMEM((_B_PER_W, N_EMBED), jnp.float32),
        pltpu.VMEM((_B_PER_W, N_EMBED), jnp.float32),
        pltpu.SemaphoreType.DMA,
    ],
    compiler_params=pltpu.CompilerParams(use_tc_tiling_on_sc=False),
)
def _gather3(in_hbm, out_hbm, self_hbm, iw_hbm, ow_hbm, w_hbm,
             o_in, o_out, o_self,
             idx1, idx2, idx3, r1, r2, r3, sem):
    wid = lax.axis_index("s") * _NC + lax.axis_index("c")
    base = wid * _B_PER_W
    pltpu.sync_copy(iw_hbm.at[pl.ds(base, _B_PER_W)], idx1)
    pltpu.sync_copy(ow_hbm.at[pl.ds(base, _B_PER_W)], idx2)
    pltpu.sync_copy(w_hbm.at[pl.ds(base, _B_PER_W)], idx3)
    copies = []
    for tbl, idx, rows in ((in_hbm, idx1, r1),
                           (out_hbm, idx2, r2),
                           (self_hbm, idx3, r3)):
        for j in range(_N_CHUNK):
            copies.append(pltpu.async_copy(
                tbl.at[idx.at[pl.ds(j * _CHUNK, _CHUNK)]],
                rows.at[pl.ds(j * _CHUNK, _CHUNK)],
                sem))
    for c in copies:
        c.wait()
    pltpu.sync_copy(r1, o_in.at[pl.ds(base, _B_PER_W)])
    pltpu.sync_copy(r2, o_out.at[pl.ds(base, _B_PER_W)])
    pltpu.sync_copy(r3, o_self.at[pl.ds(base, _B_PER_W)])


def kernel(self_embed, in_embed, out_embed, input_words, output_words, words):
    iw = input_words.astype(jnp.int32)
    ow = output_words.astype(jnp.int32)
    w = words.astype(jnp.int32)
    o_in, o_out, o_self = _gather3(in_embed, out_embed, self_embed, iw, ow, w)
    return (o_in, o_out, o_self)


# speedup vs baseline: 1.4479x; 1.4479x over previous
import functools

import jax
import jax.numpy as jnp
from jax import lax
from jax.experimental import pallas as pl
from jax.experimental.pallas import tpu as pltpu
from jax.experimental.pallas import tpu_sc as plsc

N_VOCAB = 1000000
N_EMBED = 32
BATCH = 16384

_info = plsc.get_sparse_core_info()
_NC = _info.num_cores
_NS = _info.num_subcores
_NW = _NC * _NS
_B_PER_W = BATCH // _NW
_UNROLL = 16


@functools.partial(
    pl.kernel,
    mesh=plsc.VectorSubcoreMesh(core_axis_name="c", subcore_axis_name="s"),
    out_type=[
        jax.ShapeDtypeStruct((BATCH, N_EMBED), jnp.float32),
        jax.ShapeDtypeStruct((BATCH, N_EMBED), jnp.float32),
        jax.ShapeDtypeStruct((BATCH, N_EMBED), jnp.float32),
    ],
    scratch_types=[
        pltpu.VMEM((_B_PER_W,), jnp.int32),
        pltpu.VMEM((_B_PER_W, N_EMBED), jnp.float32),
        pltpu.SemaphoreType.DMA,
    ],
)
def _gather3(in_hbm, out_hbm, self_hbm, iw_hbm, ow_hbm, w_hbm,
             o_in, o_out, o_self,
             idx_v, rows_v, sem):
    wid = lax.axis_index("s") * _NC + lax.axis_index("c")
    base = wid * _B_PER_W
    for tbl, idxh, outh in ((in_hbm, iw_hbm, o_in),
                            (out_hbm, ow_hbm, o_out),
                            (self_hbm, w_hbm, o_self)):
        pltpu.sync_copy(idxh.at[pl.ds(base, _B_PER_W)], idx_v)

        def fire(i, _):
            v = idx_v[pl.ds(i * _UNROLL, _UNROLL)]
            for u in range(_UNROLL):
                pltpu.async_copy(tbl.at[v[u]], rows_v.at[i * _UNROLL + u], sem)
            return 0

        lax.fori_loop(0, _B_PER_W // _UNROLL, fire, 0)
        pltpu.make_async_copy(tbl.at[pl.ds(0, _B_PER_W)], rows_v, sem).wait()
        pltpu.sync_copy(rows_v, outh.at[pl.ds(base, _B_PER_W)])


def kernel(self_embed, in_embed, out_embed, input_words, output_words, words):
    iw = input_words.astype(jnp.int32)
    ow = output_words.astype(jnp.int32)
    w = words.astype(jnp.int32)
    o_in, o_out, o_self = _gather3(in_embed, out_embed, self_embed, iw, ow, w)
    return (o_in, o_out, o_self)


# speedup vs baseline: 1.4496x; 1.0012x over previous
import functools

import jax
import jax.numpy as jnp
from jax import lax
from jax.experimental import pallas as pl
from jax.experimental.pallas import tpu as pltpu
from jax.experimental.pallas import tpu_sc as plsc

N_VOCAB = 1000000
N_EMBED = 32
BATCH = 16384

_info = plsc.get_sparse_core_info()
_NC = _info.num_cores
_NS = _info.num_subcores
_NW = _NC * _NS
_B_PER_W = BATCH // _NW
_UNROLL = 16
_NSEM = 4


@functools.partial(
    pl.kernel,
    mesh=plsc.VectorSubcoreMesh(core_axis_name="c", subcore_axis_name="s"),
    out_type=[
        jax.ShapeDtypeStruct((BATCH, N_EMBED), jnp.float32),
        jax.ShapeDtypeStruct((BATCH, N_EMBED), jnp.float32),
        jax.ShapeDtypeStruct((BATCH, N_EMBED), jnp.float32),
    ],
    scratch_types=[
        pltpu.VMEM((_B_PER_W,), jnp.int32),
        pltpu.VMEM((_B_PER_W, N_EMBED), jnp.float32),
        pltpu.SemaphoreType.DMA,
        pltpu.SemaphoreType.DMA,
        pltpu.SemaphoreType.DMA,
        pltpu.SemaphoreType.DMA,
    ],
    compiler_params=pltpu.CompilerParams(needs_layout_passes=False),
)
def _gather3(in_hbm, out_hbm, self_hbm, iw_hbm, ow_hbm, w_hbm,
             o_in, o_out, o_self,
             idx_v, rows_v, sem0, sem1, sem2, sem3):
    wid = lax.axis_index("s") * _NC + lax.axis_index("c")
    base = wid * _B_PER_W
    sems = (sem0, sem1, sem2, sem3)
    for tbl, idxh, outh in ((in_hbm, iw_hbm, o_in),
                            (out_hbm, ow_hbm, o_out),
                            (self_hbm, w_hbm, o_self)):
        pltpu.sync_copy(idxh.at[pl.ds(base, _B_PER_W)], idx_v)

        def fire(i, _):
            v = idx_v[pl.ds(i * _UNROLL, _UNROLL)]
            for u in range(_UNROLL):
                pltpu.async_copy(tbl.at[v[u]], rows_v.at[i * _UNROLL + u],
                                 sems[u % _NSEM])
            return 0

        lax.fori_loop(0, _B_PER_W // _UNROLL, fire, 0)
        for k in range(_NSEM):
            pltpu.make_async_copy(
                tbl.at[pl.ds(0, _B_PER_W // _NSEM)],
                rows_v.at[pl.ds(0, _B_PER_W // _NSEM)], sems[k]).wait()
        pltpu.sync_copy(rows_v, outh.at[pl.ds(base, _B_PER_W)])


def kernel(self_embed, in_embed, out_embed, input_words, output_words, words):
    iw = input_words.astype(jnp.int32)
    ow = output_words.astype(jnp.int32)
    w = words.astype(jnp.int32)
    o_in, o_out, o_self = _gather3(in_embed, out_embed, self_embed, iw, ow, w)
    return (o_in, o_out, o_self)
